# Initial kernel scaffold; baseline (speedup 1.0000x reference)
import jax
import jax.numpy as jnp
from jax.experimental import pallas as pl


def kernel(x, W_conv, b_conv):
    raise NotImplementedError("write your pallas kernel here")


# speedup vs baseline: 4.1049x; 4.1049x over previous
import functools

import jax
import jax.numpy as jnp
from jax import lax
from jax.experimental import pallas as pl
from jax.experimental.pallas import tpu as pltpu
from jax.experimental.pallas import tpu_sc as plsc

B, H, W, C = 2, 224, 224, 96
CO = 2 * C
HW = H * W
PW = W + 2
PH = H + 2
PWC = 256
FLAT_VALID = H * PWC
PT = 6400
NSTEPS = 9
XT = PT + 2 * PWC + 128
FLAT_IN = (NSTEPS - 1) * PT + XT + 128

NPLANES = B * C
NWORKERS = 32
PLANES_PER_W = NPLANES // NWORKERS
PX = 3136
NCHUNK = HW // PX
NVEC = PX // 16
VPR = W // 16


def _conv_body(x_ref, w_ref, b_ref, out_ref):
    s = pl.program_id(1)
    xt = x_ref[0, :, pl.ds(s * PT, XT)]
    acc = jnp.zeros((CO, PT), dtype=jnp.float32)
    for kx in range(3):
        xk = xt if kx == 0 else pltpu.roll(xt, XT - kx, 1)
        for ky in range(3):
            xs = xk[:, ky * PWC:ky * PWC + PT]
            wt = w_ref[ky * 3 + kx]
            acc = acc + lax.dot_general(
                wt, xs, (((1,), (0,)), ((), ())),
                preferred_element_type=jnp.float32,
                precision=lax.Precision.HIGHEST)
    out_ref[0] = acc + b_ref[...]


def _offsets_conv(x_flat, w_t, bias_col):
    return pl.pallas_call(
        _conv_body,
        grid=(B, NSTEPS),
        in_specs=[
            pl.BlockSpec((1, C, FLAT_IN), lambda b, s: (b, 0, 0)),
            pl.BlockSpec((9, CO, C), lambda b, s: (0, 0, 0)),
            pl.BlockSpec((CO, 1), lambda b, s: (0, 0)),
        ],
        out_specs=pl.BlockSpec((1, CO, PT), lambda b, s: (b, 0, s)),
        out_shape=jax.ShapeDtypeStruct((B, CO, NSTEPS * PT), jnp.float32),
        compiler_params=pltpu.CompilerParams(
            vmem_limit_bytes=100 * 1024 * 1024),
    )(x_flat, w_t, bias_col)


def _sample_body(planes_hbm, offs_hbm, out_hbm, plane_v, offs_v, out_v):
    wid = lax.axis_index("s") * 2 + lax.axis_index("c")
    iota = lax.iota(jnp.int32, 16)
    iota_f = iota.astype(jnp.float32)
    idx_even = iota * 2

    def per_plane(t, _):
        i = wid * PLANES_PER_W + t
        pltpu.sync_copy(planes_hbm.at[i], plane_v)

        def per_chunk(k, _):
            pltpu.sync_copy(offs_hbm.at[i, pl.ds(k * (2 * PX), 2 * PX)], offs_v)

            def per_vec(v, _):
                row = k * (PX // W) + v // VPR
                colb = (v % VPR) * 16
                base = v * 32
                i0 = base + idx_even
                o0 = plsc.load_gather(offs_v, [i0])
                o1 = plsc.load_gather(offs_v, [i0 + 1])
                c0 = o0 + row.astype(jnp.float32)
                c1 = o1 + (colb.astype(jnp.float32) + iota_f)
                c0 = jnp.minimum(jnp.maximum(c0, 0.0), jnp.float32(W - 1))
                c1 = jnp.minimum(jnp.maximum(c1, 0.0), jnp.float32(H - 1))
                l0 = c0.astype(jnp.int32)
                l1 = c1.astype(jnp.int32)
                l0f = l0.astype(jnp.float32)
                l1f = l1.astype(jnp.float32)
                r0 = jnp.where(c0 > l0f, l0 + 1, l0)
                r1 = jnp.where(c1 > l1f, l1 + 1, l1)
                f0 = c0 - l0f
                f1 = c1 - l1f
                one = jnp.ones((16,), jnp.int32)
                lt = plsc.load_gather(plane_v, [l0 + one, l1 + one])
                rb = plsc.load_gather(plane_v, [r0 + one, r1 + one])
                lb = plsc.load_gather(plane_v, [l0 + one, r1 + one])
                rt = plsc.load_gather(plane_v, [r0 + one, l1 + one])
                vt = lt + (rt - lt) * f0
                vb = lb + (rb - lb) * f0
                out_v[pl.ds(v * 16, 16)] = vt + (vb - vt) * f1
                return 0

            lax.fori_loop(0, NVEC, per_vec, 0)
            pltpu.sync_copy(out_v, out_hbm.at[i, pl.ds(k * PX, PX)])
            return 0

        lax.fori_loop(0, NCHUNK, per_chunk, 0)
        return 0

    lax.fori_loop(0, PLANES_PER_W, per_plane, 0)


@functools.cache
def _sample():
    return pl.kernel(
        _sample_body,
        out_type=jax.ShapeDtypeStruct((NPLANES, HW), jnp.float32),
        mesh=plsc.VectorSubcoreMesh(core_axis_name="c", subcore_axis_name="s"),
        scratch_types=[
            pltpu.VMEM((PH + 2, PW), jnp.float32),
            pltpu.VMEM((2 * PX,), jnp.float32),
            pltpu.VMEM((PX,), jnp.float32),
        ],
        compiler_params=pltpu.CompilerParams(use_tc_tiling_on_sc=False,
                                             needs_layout_passes=False),
    )


def kernel(x, W_conv, b_conv):
    x_chw = jnp.transpose(x, (0, 3, 1, 2))
    x_wide = jnp.pad(x_chw, ((0, 0), (0, 0), (1, 1), (1, PWC - W - 1)))
    x_flat = jnp.pad(x_wide.reshape(B, C, PH * PWC),
                     ((0, 0), (0, 0), (0, FLAT_IN - PH * PWC)))
    w_t = jnp.transpose(W_conv, (0, 1, 3, 2)).reshape(9, CO, C)
    bias_col = b_conv.reshape(CO, 1)

    conv = _offsets_conv(x_flat, w_t, bias_col)
    conv = conv[:, :, :FLAT_VALID].reshape(B, CO, H, PWC)[:, :, :, :W]
    offs = conv.reshape(NPLANES, 2 * HW)

    planes = jnp.pad(x_chw, ((0, 0), (0, 0), (1, 3), (1, 1)))
    planes = planes.reshape(NPLANES, PH + 2, PW)

    mapped = _sample()(planes, offs)
    return jnp.transpose(mapped.reshape(B, C, H, W), (0, 2, 3, 1))


# speedup vs baseline: 5.6864x; 1.3853x over previous
import functools

import jax
import jax.numpy as jnp
from jax import lax
from jax.experimental import pallas as pl
from jax.experimental.pallas import tpu as pltpu
from jax.experimental.pallas import tpu_sc as plsc

B, H, W, C = 2, 224, 224, 96
CO = 2 * C
HW = H * W
PW = W + 2
PH = H + 2
PWC = 256
FLAT_VALID = H * PWC
PT = 6400
NSTEPS = 9
XT = PT + 2 * PWC + 128
FLAT_IN = (NSTEPS - 1) * PT + XT + 128

NPLANES = B * C
NWORKERS = 32
PLANES_PER_W = NPLANES // NWORKERS
PX = 3136
NCHUNK = HW // PX
NVEC = PX // 16
VPR = W // 16


def _conv_body(x_ref, w_ref, b_ref, out_ref):
    s = pl.program_id(1)
    xt = x_ref[0, :, pl.ds(s * PT, XT)]
    acc = jnp.zeros((CO, PT), dtype=jnp.float32)
    for kx in range(3):
        xk = xt if kx == 0 else pltpu.roll(xt, XT - kx, 1)
        for ky in range(3):
            xs = xk[:, ky * PWC:ky * PWC + PT]
            wt = w_ref[ky * 3 + kx]
            acc = acc + lax.dot_general(
                wt, xs, (((1,), (0,)), ((), ())),
                preferred_element_type=jnp.float32,
                precision=lax.Precision.DEFAULT)
    out_ref[0] = acc + b_ref[...]


def _offsets_conv(x_flat, w_t, bias_col):
    return pl.pallas_call(
        _conv_body,
        grid=(B, NSTEPS),
        in_specs=[
            pl.BlockSpec((1, C, FLAT_IN), lambda b, s: (b, 0, 0)),
            pl.BlockSpec((9, CO, C), lambda b, s: (0, 0, 0)),
            pl.BlockSpec((CO, 1), lambda b, s: (0, 0)),
        ],
        out_specs=pl.BlockSpec((1, CO, PT), lambda b, s: (b, 0, s)),
        out_shape=jax.ShapeDtypeStruct((B, CO, NSTEPS * PT), jnp.float32),
        compiler_params=pltpu.CompilerParams(
            vmem_limit_bytes=100 * 1024 * 1024),
    )(x_flat, w_t, bias_col)


def _sample_body(planes_hbm, offs_hbm, out_hbm, plane_v, offs_v, out_v):
    wid = lax.axis_index("s") * 2 + lax.axis_index("c")
    iota = lax.iota(jnp.int32, 16)
    iota_f = iota.astype(jnp.float32)
    idx_even = iota * 2

    def per_plane(t, _):
        i = wid * PLANES_PER_W + t
        pltpu.sync_copy(planes_hbm.at[i], plane_v)

        def per_chunk(k, _):
            pltpu.sync_copy(offs_hbm.at[i, pl.ds(k * (2 * PX), 2 * PX)], offs_v)

            def per_vec(v, _):
                row = k * (PX // W) + v // VPR
                colb = (v % VPR) * 16
                base = v * 32
                i0 = base + idx_even
                o0 = plsc.load_gather(offs_v, [i0])
                o1 = plsc.load_gather(offs_v, [i0 + 1])
                c0 = o0 + row.astype(jnp.float32)
                c1 = o1 + (colb.astype(jnp.float32) + iota_f)
                c0 = jnp.minimum(jnp.maximum(c0, 0.0), jnp.float32(W - 1))
                c1 = jnp.minimum(jnp.maximum(c1, 0.0), jnp.float32(H - 1))
                l0 = c0.astype(jnp.int32)
                l1 = c1.astype(jnp.int32)
                l0f = l0.astype(jnp.float32)
                l1f = l1.astype(jnp.float32)
                r0 = jnp.where(c0 > l0f, l0 + 1, l0)
                r1 = jnp.where(c1 > l1f, l1 + 1, l1)
                f0 = c0 - l0f
                f1 = c1 - l1f
                one = jnp.ones((16,), jnp.int32)
                lt = plsc.load_gather(plane_v, [l0 + one, l1 + one])
                rb = plsc.load_gather(plane_v, [r0 + one, r1 + one])
                lb = plsc.load_gather(plane_v, [l0 + one, r1 + one])
                rt = plsc.load_gather(plane_v, [r0 + one, l1 + one])
                vt = lt + (rt - lt) * f0
                vb = lb + (rb - lb) * f0
                out_v[pl.ds(v * 16, 16)] = vt + (vb - vt) * f1
                return 0

            lax.fori_loop(0, NVEC, per_vec, 0)
            pltpu.sync_copy(out_v, out_hbm.at[i, pl.ds(k * PX, PX)])
            return 0

        lax.fori_loop(0, NCHUNK, per_chunk, 0)
        return 0

    lax.fori_loop(0, PLANES_PER_W, per_plane, 0)


@functools.cache
def _sample():
    return pl.kernel(
        _sample_body,
        out_type=jax.ShapeDtypeStruct((NPLANES, HW), jnp.float32),
        mesh=plsc.VectorSubcoreMesh(core_axis_name="c", subcore_axis_name="s"),
        scratch_types=[
            pltpu.VMEM((PH + 2, PW), jnp.float32),
            pltpu.VMEM((2 * PX,), jnp.float32),
            pltpu.VMEM((PX,), jnp.float32),
        ],
        compiler_params=pltpu.CompilerParams(use_tc_tiling_on_sc=False,
                                             needs_layout_passes=False),
    )


def kernel(x, W_conv, b_conv):
    x_chw = jnp.transpose(x, (0, 3, 1, 2))
    x_wide = jnp.pad(x_chw, ((0, 0), (0, 0), (1, 1), (1, PWC - W - 1)))
    x_flat = jnp.pad(x_wide.reshape(B, C, PH * PWC),
                     ((0, 0), (0, 0), (0, FLAT_IN - PH * PWC)))
    w_t = jnp.transpose(W_conv, (0, 1, 3, 2)).reshape(9, CO, C)
    bias_col = b_conv.reshape(CO, 1)

    conv = _offsets_conv(x_flat, w_t, bias_col)
    conv = conv[:, :, :FLAT_VALID].reshape(B, CO, H, PWC)[:, :, :, :W]
    offs = conv.reshape(NPLANES, 2 * HW)

    planes = jnp.pad(x_chw, ((0, 0), (0, 0), (1, 3), (1, 1)))
    planes = planes.reshape(NPLANES, PH + 2, PW)

    mapped = _sample()(planes, offs)
    return jnp.transpose(mapped.reshape(B, C, H, W), (0, 2, 3, 1))


# speedup vs baseline: 6.2256x; 1.0948x over previous
import functools

import jax
import jax.numpy as jnp
from jax import lax
from jax.experimental import pallas as pl
from jax.experimental.pallas import tpu as pltpu
from jax.experimental.pallas import tpu_sc as plsc

B, H, W, C = 2, 224, 224, 96
CO = 2 * C
HW = H * W
PW = W + 2
PH = H + 2
PWC = 256
FLAT_VALID = H * PWC
PT = 6400
NSTEPS = 9
XT = PT + 2 * PWC + 128
FLAT_IN = (NSTEPS - 1) * PT + XT + 128

NPLANES = B * C
NWORKERS = 32
PLANES_PER_W = NPLANES // NWORKERS
PX = 6272
NCHUNK = HW // PX
NVEC = PX // 16
VPR = W // 16
RPC = PX // W


def _conv_body(x_ref, w_ref, b_ref, out_ref):
    s = pl.program_id(1)
    xt = x_ref[0, :, pl.ds(s * PT, XT)]
    acc = jnp.zeros((CO, PT), dtype=jnp.float32)
    for kx in range(3):
        xk = xt if kx == 0 else pltpu.roll(xt, XT - kx, 1)
        for ky in range(3):
            xs = xk[:, ky * PWC:ky * PWC + PT]
            wt = w_ref[ky * 3 + kx]
            acc = acc + lax.dot_general(
                wt, xs, (((1,), (0,)), ((), ())),
                preferred_element_type=jnp.float32,
                precision=lax.Precision.DEFAULT)
    out_ref[0] = acc + b_ref[...]


def _offsets_conv(x_flat, w_t, bias_col):
    return pl.pallas_call(
        _conv_body,
        grid=(B, NSTEPS),
        in_specs=[
            pl.BlockSpec((1, C, FLAT_IN), lambda b, s: (b, 0, 0)),
            pl.BlockSpec((9, CO, C), lambda b, s: (0, 0, 0)),
            pl.BlockSpec((CO, 1), lambda b, s: (0, 0)),
        ],
        out_specs=pl.BlockSpec((1, CO, PT), lambda b, s: (b, 0, s)),
        out_shape=jax.ShapeDtypeStruct((B, CO, NSTEPS * PT), jnp.float32),
        compiler_params=pltpu.CompilerParams(
            vmem_limit_bytes=100 * 1024 * 1024),
    )(x_flat, w_t, bias_col)


def _sample_body(planes_hbm, offs_hbm, out_hbm, plane_v, offs_v, out_v,
                 sem_in, sem_o0, sem_o1):
    wid = lax.axis_index("s") * 2 + lax.axis_index("c")
    iota = lax.iota(jnp.int32, 16)
    iota_f = iota.astype(jnp.float32)
    idx_even = iota * 2
    sem_o = (sem_o0, sem_o1)

    def compute_chunk(k, obuf, rbuf):
        @functools.partial(lax.fori_loop, 0, NVEC, init_val=0, unroll=4)
        def _vec(v, _carry):
            rowp = (k * RPC + v // VPR + 1).astype(jnp.float32)
            colp = ((v % VPR) * 16 + 1).astype(jnp.float32)
            i0 = v * 32 + idx_even
            o0 = plsc.load_gather(obuf, [i0])
            o1 = plsc.load_gather(obuf, [i0 + 1])
            c0 = o0 + rowp
            c1 = o1 + (colp + iota_f)
            c0 = jnp.minimum(jnp.maximum(c0, 1.0), jnp.float32(W))
            c1 = jnp.minimum(jnp.maximum(c1, 1.0), jnp.float32(H))
            l0 = c0.astype(jnp.int32)
            l1 = c1.astype(jnp.int32)
            l0f = l0.astype(jnp.float32)
            l1f = l1.astype(jnp.float32)
            r0 = l0 + (c0 > l0f).astype(jnp.int32)
            r1 = l1 + (c1 > l1f).astype(jnp.int32)
            f0 = c0 - l0f
            f1 = c1 - l1f
            lt = plsc.load_gather(plane_v, [l0, l1])
            rb = plsc.load_gather(plane_v, [r0, r1])
            lb = plsc.load_gather(plane_v, [l0, r1])
            rt = plsc.load_gather(plane_v, [r0, l1])
            vt = lt + (rt - lt) * f0
            vb = lb + (rb - lb) * f0
            rbuf[pl.ds(v * 16, 16)] = vt + (vb - vt) * f1
            return 0

    def per_plane(t, _):
        i = wid * PLANES_PER_W + t
        pltpu.sync_copy(planes_hbm.at[i], plane_v)

        def chunk(k, _):
            obuf = offs_v
            rbuf = out_v
            pltpu.sync_copy(offs_hbm.at[i, pl.ds(k * 2 * PX, 2 * PX)], obuf)
            compute_chunk(k, obuf, rbuf)
            pltpu.sync_copy(rbuf, out_hbm.at[i, pl.ds(k * PX, PX)])
            return 0

        lax.fori_loop(0, NCHUNK, chunk, 0)
        return 0

    lax.fori_loop(0, PLANES_PER_W, per_plane, 0)


@functools.cache
def _sample():
    return pl.kernel(
        _sample_body,
        out_type=jax.ShapeDtypeStruct((NPLANES, HW), jnp.float32),
        mesh=plsc.VectorSubcoreMesh(core_axis_name="c", subcore_axis_name="s"),
        scratch_types=[
            pltpu.VMEM((PH + 2, PWC), jnp.float32),
            pltpu.VMEM((2 * PX,), jnp.float32),
            pltpu.VMEM((PX,), jnp.float32),
            pltpu.SemaphoreType.DMA,
            pltpu.SemaphoreType.DMA,
            pltpu.SemaphoreType.DMA,
        ],
        compiler_params=pltpu.CompilerParams(use_tc_tiling_on_sc=False,
                                             needs_layout_passes=False),
    )


def kernel(x, W_conv, b_conv):
    x_chw = jnp.transpose(x, (0, 3, 1, 2))
    x_wide = jnp.pad(x_chw, ((0, 0), (0, 0), (1, 1), (1, PWC - W - 1)))
    x_flat = jnp.pad(x_wide.reshape(B, C, PH * PWC),
                     ((0, 0), (0, 0), (0, FLAT_IN - PH * PWC)))
    w_t = jnp.transpose(W_conv, (0, 1, 3, 2)).reshape(9, CO, C)
    bias_col = b_conv.reshape(CO, 1)

    conv = _offsets_conv(x_flat, w_t, bias_col)
    conv = conv[:, :, :FLAT_VALID].reshape(B, CO, H, PWC)[:, :, :, :W]
    offs = conv.reshape(NPLANES, 2 * HW)

    planes = x_flat.reshape(NPLANES, PH + 2, PWC)

    mapped = _sample()(planes, offs)
    return jnp.transpose(mapped.reshape(B, C, H, W), (0, 2, 3, 1))


# speedup vs baseline: 6.5664x; 1.0547x over previous
import functools

import jax
import jax.numpy as jnp
from jax import lax
from jax.experimental import pallas as pl
from jax.experimental.pallas import tpu as pltpu
from jax.experimental.pallas import tpu_sc as plsc

B, H, W, C = 2, 224, 224, 96
CO = 2 * C
HW = H * W
PW = W + 2
PH = H + 2
PWC = 256
FLAT_VALID = H * PWC
PT = 6400
NSTEPS = 9
XT = PT + 2 * PWC + 128
FLAT_IN = (NSTEPS - 1) * PT + XT + 128

NPLANES = B * C
NWORKERS = 32
PLANES_PER_W = NPLANES // NWORKERS
PX = 6272
NCHUNK = HW // PX
NVEC = PX // 16
VPR = W // 16
RPC = PX // W


def _conv_body(x_ref, w_ref, b_ref, out_ref):
    s = pl.program_id(1)
    xt = x_ref[0, :, pl.ds(s * PT, XT)]
    acc = jnp.zeros((CO, PT), dtype=jnp.float32)
    for kx in range(3):
        xk = xt if kx == 0 else pltpu.roll(xt, XT - kx, 1)
        for ky in range(3):
            xs = xk[:, ky * PWC:ky * PWC + PT]
            wt = w_ref[ky * 3 + kx]
            acc = acc + lax.dot_general(
                wt, xs, (((1,), (0,)), ((), ())),
                preferred_element_type=jnp.float32,
                precision=lax.Precision.DEFAULT)
    out_ref[0] = acc + b_ref[...]


def _offsets_conv(x_flat, w_t, bias_col):
    return pl.pallas_call(
        _conv_body,
        grid=(B, NSTEPS),
        in_specs=[
            pl.BlockSpec((1, C, FLAT_IN), lambda b, s: (b, 0, 0)),
            pl.BlockSpec((9, CO, C), lambda b, s: (0, 0, 0)),
            pl.BlockSpec((CO, 1), lambda b, s: (0, 0)),
        ],
        out_specs=pl.BlockSpec((1, CO, PT), lambda b, s: (b, 0, s)),
        out_shape=jax.ShapeDtypeStruct((B, CO, NSTEPS * PT), jnp.float32),
        compiler_params=pltpu.CompilerParams(
            vmem_limit_bytes=100 * 1024 * 1024),
    )(x_flat, w_t, bias_col)


def _sample_body(planes_hbm, offs_hbm, out_hbm, plane_v, offs_v, out_v,
                 sem_in, sem_o0, sem_o1):
    wid = lax.axis_index("s") * 2 + lax.axis_index("c")
    iota = lax.iota(jnp.int32, 16)
    iota_f = iota.astype(jnp.float32)
    idx_even = iota * 2
    sem_o = (sem_o0, sem_o1)

    def compute_chunk(k, obuf, rbuf):
        @functools.partial(lax.fori_loop, 0, NVEC, init_val=0, unroll=4)
        def _vec(v, _carry):
            rowp = (k * RPC + v // VPR + 1).astype(jnp.float32)
            colp = ((v % VPR) * 16 + 1).astype(jnp.float32)
            i0 = v * 32 + idx_even
            o0 = plsc.load_gather(obuf, [i0])
            o1 = plsc.load_gather(obuf, [i0 + 1])
            c0 = o0 + rowp
            c1 = o1 + (colp + iota_f)
            c0 = jnp.minimum(jnp.maximum(c0, 1.0), jnp.float32(W))
            c1 = jnp.minimum(jnp.maximum(c1, 1.0), jnp.float32(H))
            l0 = c0.astype(jnp.int32)
            l1 = c1.astype(jnp.int32)
            l0f = l0.astype(jnp.float32)
            l1f = l1.astype(jnp.float32)
            r0 = l0 + (c0 > l0f).astype(jnp.int32)
            r1 = l1 + (c1 > l1f).astype(jnp.int32)
            f0 = c0 - l0f
            f1 = c1 - l1f
            lt = plsc.load_gather(plane_v, [l0, l1])
            rb = plsc.load_gather(plane_v, [r0, r1])
            lb = plsc.load_gather(plane_v, [l0, r1])
            rt = plsc.load_gather(plane_v, [r0, l1])
            vt = lt + (rt - lt) * f0
            vb = lb + (rb - lb) * f0
            rbuf[pl.ds(v * 16, 16)] = vt + (vb - vt) * f1
            return 0

    def per_plane(t, _):
        i = wid * PLANES_PER_W + t
        pltpu.sync_copy(planes_hbm.at[i], plane_v)
        pltpu.async_copy(offs_hbm.at[i, pl.ds(0, 2 * PX)], offs_v.at[0], sem_in)

        def pair(k2, _):
            for par in (0, 1):
                k = k2 + par
                obuf = offs_v.at[par]
                rbuf = out_v.at[par]
                pltpu.make_async_copy(
                    offs_hbm.at[i, pl.ds(k * 2 * PX, 2 * PX)], obuf, sem_in
                ).wait()

                @pl.when(k + 1 < NCHUNK)
                def _():
                    pltpu.async_copy(
                        offs_hbm.at[i, pl.ds((k + 1) * 2 * PX, 2 * PX)],
                        offs_v.at[1 - par], sem_in)

                @pl.when(k >= 2)
                def _():
                    pltpu.make_async_copy(
                        rbuf, out_hbm.at[i, pl.ds((k - 2) * PX, PX)], sem_o[par]
                    ).wait()

                compute_chunk(k, obuf, rbuf)
                pltpu.async_copy(
                    rbuf, out_hbm.at[i, pl.ds(k * PX, PX)], sem_o[par])
            return 0

        lax.fori_loop(0, NCHUNK // 2, lambda j, c: pair(2 * j, c), 0)
        pltpu.make_async_copy(
            out_v.at[0], out_hbm.at[i, pl.ds((NCHUNK - 2) * PX, PX)], sem_o[0]
        ).wait()
        pltpu.make_async_copy(
            out_v.at[1], out_hbm.at[i, pl.ds((NCHUNK - 1) * PX, PX)], sem_o[1]
        ).wait()
        return 0

    lax.fori_loop(0, PLANES_PER_W, per_plane, 0)


@functools.cache
def _sample():
    return pl.kernel(
        _sample_body,
        out_type=jax.ShapeDtypeStruct((NPLANES, HW), jnp.float32),
        mesh=plsc.VectorSubcoreMesh(core_axis_name="c", subcore_axis_name="s"),
        scratch_types=[
            pltpu.VMEM((PH + 2, PWC), jnp.float32),
            pltpu.VMEM((2, 2 * PX), jnp.float32),
            pltpu.VMEM((2, PX), jnp.float32),
            pltpu.SemaphoreType.DMA,
            pltpu.SemaphoreType.DMA,
            pltpu.SemaphoreType.DMA,
        ],
        compiler_params=pltpu.CompilerParams(use_tc_tiling_on_sc=False,
                                             needs_layout_passes=False),
    )


def kernel(x, W_conv, b_conv):
    x_chw = jnp.transpose(x, (0, 3, 1, 2))
    x_wide = jnp.pad(x_chw, ((0, 0), (0, 0), (1, 1), (1, PWC - W - 1)))
    x_flat = jnp.pad(x_wide.reshape(B, C, PH * PWC),
                     ((0, 0), (0, 0), (0, FLAT_IN - PH * PWC)))
    w_t = jnp.transpose(W_conv, (0, 1, 3, 2)).reshape(9, CO, C)
    bias_col = b_conv.reshape(CO, 1)

    conv = _offsets_conv(x_flat, w_t, bias_col)
    conv = conv[:, :, :FLAT_VALID].reshape(B, CO, H, PWC)[:, :, :, :W]
    offs = conv.reshape(NPLANES, 2 * HW)

    planes = x_flat.reshape(NPLANES, PH + 2, PWC)

    mapped = _sample()(planes, offs)
    return jnp.transpose(mapped.reshape(B, C, H, W), (0, 2, 3, 1))


# speedup vs baseline: 6.6590x; 1.0141x over previous
import functools

import jax
import jax.numpy as jnp
from jax import lax
from jax.experimental import pallas as pl
from jax.experimental.pallas import tpu as pltpu
from jax.experimental.pallas import tpu_sc as plsc

B, H, W, C = 2, 224, 224, 96
CO = 2 * C
HW = H * W
PW = W + 2
PH = H + 2
PWC = 256
FLAT_VALID = H * PWC
PT = 6400
NSTEPS = 9
XT = PT + 2 * PWC + 128
FLAT_IN = (NSTEPS - 1) * PT + XT + 128

NPLANES = B * C
NWORKERS = 32
PLANES_PER_W = NPLANES // NWORKERS
PX = 6272
NCHUNK = HW // PX
NVEC = PX // 16
VPR = W // 16
RPC = PX // W


def _conv_body(x_ref, w_ref, b_ref, out_ref):
    s = pl.program_id(1)
    xt = x_ref[0, :, pl.ds(s * PT, XT)]
    acc = jnp.zeros((CO, PT), dtype=jnp.float32)
    for kx in range(3):
        xk = xt if kx == 0 else pltpu.roll(xt, XT - kx, 1)
        for ky in range(3):
            xs = xk[:, ky * PWC:ky * PWC + PT]
            wt = w_ref[ky * 3 + kx]
            acc = acc + lax.dot_general(
                wt, xs, (((1,), (0,)), ((), ())),
                preferred_element_type=jnp.float32,
                precision=lax.Precision.DEFAULT)
    out_ref[0] = acc + b_ref[...]


def _offsets_conv(x_flat, w_t, bias_col):
    return pl.pallas_call(
        _conv_body,
        grid=(B, NSTEPS),
        in_specs=[
            pl.BlockSpec((1, C, FLAT_IN), lambda b, s: (b, 0, 0)),
            pl.BlockSpec((9, CO, C), lambda b, s: (0, 0, 0)),
            pl.BlockSpec((CO, 1), lambda b, s: (0, 0)),
        ],
        out_specs=pl.BlockSpec((1, CO, PT), lambda b, s: (b, 0, s)),
        out_shape=jax.ShapeDtypeStruct((B, CO, NSTEPS * PT), jnp.float32),
        compiler_params=pltpu.CompilerParams(
            vmem_limit_bytes=100 * 1024 * 1024),
    )(x_flat, w_t, bias_col)


def _sample_body(planes_hbm, offs_hbm, out_hbm, plane_v, offs_v, out_v,
                 sem_in, sem_o0, sem_o1):
    wid = lax.axis_index("s") * 2 + lax.axis_index("c")
    iota = lax.iota(jnp.int32, 16)
    iota_f = iota.astype(jnp.float32)
    idx_even = iota * 2
    sem_o = (sem_o0, sem_o1)

    def compute_chunk(k, obuf, rbuf):
        @functools.partial(lax.fori_loop, 0, NVEC, init_val=0, unroll=8)
        def _vec(v, _carry):
            rowp = (k * RPC + v // VPR + 1).astype(jnp.float32)
            colp = ((v % VPR) * 16 + 1).astype(jnp.float32)
            i0 = v * 32 + idx_even
            o0 = plsc.load_gather(obuf, [i0])
            o1 = plsc.load_gather(obuf, [i0 + 1])
            c0 = o0 + rowp
            c1 = o1 + (colp + iota_f)
            c0 = jnp.minimum(jnp.maximum(c0, 1.0), jnp.float32(W))
            c1 = jnp.minimum(jnp.maximum(c1, 1.0), jnp.float32(H))
            l0 = c0.astype(jnp.int32)
            l1 = c1.astype(jnp.int32)
            l0f = l0.astype(jnp.float32)
            l1f = l1.astype(jnp.float32)
            r0 = l0 + (c0 > l0f).astype(jnp.int32)
            r1 = l1 + (c1 > l1f).astype(jnp.int32)
            f0 = c0 - l0f
            f1 = c1 - l1f
            lt = plsc.load_gather(plane_v, [l0, l1])
            rb = plsc.load_gather(plane_v, [r0, r1])
            lb = plsc.load_gather(plane_v, [l0, r1])
            rt = plsc.load_gather(plane_v, [r0, l1])
            vt = lt + (rt - lt) * f0
            vb = lb + (rb - lb) * f0
            rbuf[pl.ds(v * 16, 16)] = vt + (vb - vt) * f1
            return 0

    def per_plane(t, _):
        i = wid * PLANES_PER_W + t
        pltpu.sync_copy(planes_hbm.at[i], plane_v)
        pltpu.async_copy(offs_hbm.at[i, pl.ds(0, 2 * PX)], offs_v.at[0], sem_in)

        def pair(k2, _):
            for par in (0, 1):
                k = k2 + par
                obuf = offs_v.at[par]
                rbuf = out_v.at[par]
                pltpu.make_async_copy(
                    offs_hbm.at[i, pl.ds(k * 2 * PX, 2 * PX)], obuf, sem_in
                ).wait()

                @pl.when(k + 1 < NCHUNK)
                def _():
                    pltpu.async_copy(
                        offs_hbm.at[i, pl.ds((k + 1) * 2 * PX, 2 * PX)],
                        offs_v.at[1 - par], sem_in)

                @pl.when(k >= 2)
                def _():
                    pltpu.make_async_copy(
                        rbuf, out_hbm.at[i, pl.ds((k - 2) * PX, PX)], sem_o[par]
                    ).wait()

                compute_chunk(k, obuf, rbuf)
                pltpu.async_copy(
                    rbuf, out_hbm.at[i, pl.ds(k * PX, PX)], sem_o[par])
            return 0

        lax.fori_loop(0, NCHUNK // 2, lambda j, c: pair(2 * j, c), 0)
        pltpu.make_async_copy(
            out_v.at[0], out_hbm.at[i, pl.ds((NCHUNK - 2) * PX, PX)], sem_o[0]
        ).wait()
        pltpu.make_async_copy(
            out_v.at[1], out_hbm.at[i, pl.ds((NCHUNK - 1) * PX, PX)], sem_o[1]
        ).wait()
        return 0

    lax.fori_loop(0, PLANES_PER_W, per_plane, 0)


@functools.cache
def _sample():
    return pl.kernel(
        _sample_body,
        out_type=jax.ShapeDtypeStruct((NPLANES, HW), jnp.float32),
        mesh=plsc.VectorSubcoreMesh(core_axis_name="c", subcore_axis_name="s"),
        scratch_types=[
            pltpu.VMEM((PH + 2, PWC), jnp.float32),
            pltpu.VMEM((2, 2 * PX), jnp.float32),
            pltpu.VMEM((2, PX), jnp.float32),
            pltpu.SemaphoreType.DMA,
            pltpu.SemaphoreType.DMA,
            pltpu.SemaphoreType.DMA,
        ],
        compiler_params=pltpu.CompilerParams(use_tc_tiling_on_sc=False,
                                             needs_layout_passes=False),
    )


def kernel(x, W_conv, b_conv):
    x_chw = jnp.transpose(x, (0, 3, 1, 2))
    x_wide = jnp.pad(x_chw, ((0, 0), (0, 0), (1, 1), (1, PWC - W - 1)))
    x_flat = jnp.pad(x_wide.reshape(B, C, PH * PWC),
                     ((0, 0), (0, 0), (0, FLAT_IN - PH * PWC)))
    w_t = jnp.transpose(W_conv, (0, 1, 3, 2)).reshape(9, CO, C)
    bias_col = b_conv.reshape(CO, 1)

    conv = _offsets_conv(x_flat, w_t, bias_col)
    conv = conv[:, :, :FLAT_VALID].reshape(B, CO, H, PWC)[:, :, :, :W]
    offs = conv.reshape(NPLANES, 2 * HW)

    planes = x_flat.reshape(NPLANES, PH + 2, PWC)

    mapped = _sample()(planes, offs)
    return jnp.transpose(mapped.reshape(B, C, H, W), (0, 2, 3, 1))


# speedup vs baseline: 7.5787x; 1.1381x over previous
import functools

import jax
import jax.numpy as jnp
from jax import lax
from jax.experimental import pallas as pl
from jax.experimental.pallas import tpu as pltpu
from jax.experimental.pallas import tpu_sc as plsc

B, H, W, C = 2, 224, 224, 96
CO = 2 * C
HW = H * W
PW = W + 2
PH = H + 2
PWC = 256
FLAT_VALID = H * PWC
PT = 6400
NSTEPS = 9
XT = PT + 2 * PWC + 128
FLAT_IN = (NSTEPS - 1) * PT + XT + 128

NPLANES = B * C
NWORKERS = 32
PLANES_PER_W = NPLANES // NWORKERS
PX = 6272
NCHUNK = HW // PX
NVEC = PX // 16
VPR = W // 16
RPC = PX // W


def _conv_body(x_ref, w_ref, b_ref, out_ref):
    s = pl.program_id(1)
    xt = x_ref[0, :, pl.ds(s * PT, XT)]
    acc = jnp.zeros((CO, PT), dtype=jnp.float32)
    for kx in range(3):
        xk = xt if kx == 0 else pltpu.roll(xt, XT - kx, 1)
        for ky in range(3):
            xs = xk[:, ky * PWC:ky * PWC + PT]
            wt = w_ref[ky * 3 + kx]
            acc = acc + lax.dot_general(
                wt, xs, (((1,), (0,)), ((), ())),
                preferred_element_type=jnp.float32,
                precision=lax.Precision.DEFAULT)
    out_ref[0] = acc + b_ref[...]


def _offsets_conv(x_flat, w_t, bias_col):
    return pl.pallas_call(
        _conv_body,
        grid=(B, NSTEPS),
        in_specs=[
            pl.BlockSpec((1, C, FLAT_IN), lambda b, s: (b, 0, 0)),
            pl.BlockSpec((9, CO, C), lambda b, s: (0, 0, 0)),
            pl.BlockSpec((CO, 1), lambda b, s: (0, 0)),
        ],
        out_specs=pl.BlockSpec((1, CO, PT), lambda b, s: (b, 0, s)),
        out_shape=jax.ShapeDtypeStruct((B, CO, NSTEPS * PT), jnp.float32),
        compiler_params=pltpu.CompilerParams(
            vmem_limit_bytes=100 * 1024 * 1024),
    )(x_flat, w_t, bias_col)


def _sample_body(planes_hbm, offs_hbm, out_hbm, plane_v, offs_v, out_v,
                 sem_in, sem_o0, sem_o1):
    wid = lax.axis_index("s") * 2 + lax.axis_index("c")
    iota = lax.iota(jnp.int32, 16)
    iota_f = iota.astype(jnp.float32)
    idx_even = iota * 2
    sem_o = (sem_o0, sem_o1)

    def compute_chunk(k, obuf, rbuf):
        @functools.partial(lax.fori_loop, 0, NVEC, init_val=0, unroll=8)
        def _vec(v, _carry):
            rowp = (k * RPC + v // VPR + 1).astype(jnp.float32)
            colp = ((v % VPR) * 16 + 1).astype(jnp.float32)
            srow = jnp.broadcast_to(v // 7, (16,))
            scol = (v % 7) * 32 + idx_even
            o0 = plsc.load_gather(obuf, [srow, scol])
            o1 = plsc.load_gather(obuf, [srow, scol + 1])
            c0 = o0 + rowp
            c1 = o1 + (colp + iota_f)
            c0 = jnp.minimum(jnp.maximum(c0, 1.0), jnp.float32(W))
            c1 = jnp.minimum(jnp.maximum(c1, 1.0), jnp.float32(H))
            l0 = c0.astype(jnp.int32)
            l1 = c1.astype(jnp.int32)
            l0f = l0.astype(jnp.float32)
            l1f = l1.astype(jnp.float32)
            r0 = l0 + (c0 > l0f).astype(jnp.int32)
            r1 = l1 + (c1 > l1f).astype(jnp.int32)
            f0 = c0 - l0f
            f1 = c1 - l1f
            lt = plsc.load_gather(plane_v, [l0, l1])
            rb = plsc.load_gather(plane_v, [r0, r1])
            lb = plsc.load_gather(plane_v, [l0, r1])
            rt = plsc.load_gather(plane_v, [r0, l1])
            vt = lt + (rt - lt) * f0
            vb = lb + (rb - lb) * f0
            rbuf[pl.ds(v * 16, 16)] = vt + (vb - vt) * f1
            return 0

    def per_plane(t, _):
        i = wid * PLANES_PER_W + t
        bb = i // C
        jj = i % C

        def offs_src(k):
            return offs_hbm.at[bb, 2 * jj + k // 4,
                               pl.ds((k % 4) * (2 * RPC), 2 * RPC),
                               pl.ds(0, W)]

        pltpu.sync_copy(planes_hbm.at[i], plane_v)
        pltpu.async_copy(offs_src(0), offs_v.at[0], sem_in)

        def pair(k2, _):
            for par in (0, 1):
                k = k2 + par
                obuf = offs_v.at[par]
                rbuf = out_v.at[par]
                pltpu.make_async_copy(offs_src(k), obuf, sem_in).wait()

                @pl.when(k + 1 < NCHUNK)
                def _():
                    pltpu.async_copy(offs_src(k + 1), offs_v.at[1 - par],
                                     sem_in)

                @pl.when(k >= 2)
                def _():
                    pltpu.make_async_copy(
                        rbuf, out_hbm.at[i, pl.ds((k - 2) * PX, PX)], sem_o[par]
                    ).wait()

                compute_chunk(k, obuf, rbuf)
                pltpu.async_copy(
                    rbuf, out_hbm.at[i, pl.ds(k * PX, PX)], sem_o[par])
            return 0

        lax.fori_loop(0, NCHUNK // 2, lambda j, c: pair(2 * j, c), 0)
        pltpu.make_async_copy(
            out_v.at[0], out_hbm.at[i, pl.ds((NCHUNK - 2) * PX, PX)], sem_o[0]
        ).wait()
        pltpu.make_async_copy(
            out_v.at[1], out_hbm.at[i, pl.ds((NCHUNK - 1) * PX, PX)], sem_o[1]
        ).wait()
        return 0

    lax.fori_loop(0, PLANES_PER_W, per_plane, 0)


@functools.cache
def _sample():
    return pl.kernel(
        _sample_body,
        out_type=jax.ShapeDtypeStruct((NPLANES, HW), jnp.float32),
        mesh=plsc.VectorSubcoreMesh(core_axis_name="c", subcore_axis_name="s"),
        scratch_types=[
            pltpu.VMEM((PH + 2, PWC), jnp.float32),
            pltpu.VMEM((2, 2 * RPC, W), jnp.float32),
            pltpu.VMEM((2, PX), jnp.float32),
            pltpu.SemaphoreType.DMA,
            pltpu.SemaphoreType.DMA,
            pltpu.SemaphoreType.DMA,
        ],
        compiler_params=pltpu.CompilerParams(use_tc_tiling_on_sc=False,
                                             needs_layout_passes=False),
    )


def kernel(x, W_conv, b_conv):
    x_chw = jnp.transpose(x, (0, 3, 1, 2))
    x_wide = jnp.pad(x_chw, ((0, 0), (0, 0), (1, 1), (1, PWC - W - 1)))
    x_flat = jnp.pad(x_wide.reshape(B, C, PH * PWC),
                     ((0, 0), (0, 0), (0, FLAT_IN - PH * PWC)))
    w_t = jnp.transpose(W_conv, (0, 1, 3, 2)).reshape(9, CO, C)
    bias_col = b_conv.reshape(CO, 1)

    conv = _offsets_conv(x_flat, w_t, bias_col)
    conv4 = conv.reshape(B, CO, NSTEPS * PT // PWC, PWC)

    planes = x_flat.reshape(NPLANES, PH + 2, PWC)

    mapped = _sample()(planes, conv4)
    return jnp.transpose(mapped.reshape(B, C, H, W), (0, 2, 3, 1))


# speedup vs baseline: 7.7180x; 1.0184x over previous
import functools

import jax
import jax.numpy as jnp
from jax import lax
from jax.experimental import pallas as pl
from jax.experimental.pallas import tpu as pltpu
from jax.experimental.pallas import tpu_sc as plsc

B, H, W, C = 2, 224, 224, 96
CO = 2 * C
HW = H * W
PW = W + 2
PH = H + 2
PWC = 256
FLAT_VALID = H * PWC
PT = 6400
NSTEPS = 9
XT = PT + 2 * PWC + 128
FLAT_IN = (NSTEPS - 1) * PT + XT + 128

NPLANES = B * C
NWORKERS = 32
PLANES_PER_W = NPLANES // NWORKERS
PX = 6272
NCHUNK = HW // PX
NVEC = PX // 16
VPR = W // 16
RPC = PX // W


def _conv_body(x_ref, w_ref, b_ref, out_ref):
    s = pl.program_id(1)
    xt = x_ref[0, :, pl.ds(s * PT, XT)]
    acc = jnp.zeros((CO, PT), dtype=jnp.float32)
    for kx in range(3):
        xk = xt if kx == 0 else pltpu.roll(xt, XT - kx, 1)
        for ky in range(3):
            xs = xk[:, ky * PWC:ky * PWC + PT]
            wt = w_ref[ky * 3 + kx]
            acc = acc + lax.dot_general(
                wt, xs, (((1,), (0,)), ((), ())),
                preferred_element_type=jnp.float32,
                precision=lax.Precision.DEFAULT)
    out_ref[0] = acc + b_ref[...]


def _offsets_conv(x_flat, w_t, bias_col):
    return pl.pallas_call(
        _conv_body,
        grid=(B, NSTEPS),
        in_specs=[
            pl.BlockSpec((1, C, FLAT_IN), lambda b, s: (b, 0, 0)),
            pl.BlockSpec((9, CO, C), lambda b, s: (0, 0, 0)),
            pl.BlockSpec((CO, 1), lambda b, s: (0, 0)),
        ],
        out_specs=pl.BlockSpec((1, CO, PT), lambda b, s: (b, 0, s)),
        out_shape=jax.ShapeDtypeStruct((B, CO, NSTEPS * PT), jnp.float32),
        compiler_params=pltpu.CompilerParams(
            vmem_limit_bytes=100 * 1024 * 1024),
    )(x_flat, w_t, bias_col)


def _sample_body(planes_hbm, offs_hbm, out_hbm, plane_v, offs_v, out_v,
                 sem_in, sem_o0, sem_o1):
    wid = lax.axis_index("s") * 2 + lax.axis_index("c")
    iota = lax.iota(jnp.int32, 16)
    iota_f = iota.astype(jnp.float32)
    idx_even = iota * 2
    sem_o = (sem_o0, sem_o1)

    def compute_chunk(k, obuf, rbuf):
        @functools.partial(lax.fori_loop, 0, NVEC, init_val=0, unroll=14)
        def _vec(v, _carry):
            rowp = (k * RPC + v // VPR + 1).astype(jnp.float32)
            colp = ((v % VPR) * 16 + 1).astype(jnp.float32)
            srow = jnp.broadcast_to(v // 7, (16,))
            scol = (v % 7) * 32 + idx_even
            o0 = plsc.load_gather(obuf, [srow, scol])
            o1 = plsc.load_gather(obuf, [srow, scol + 1])
            c0 = o0 + rowp
            c1 = o1 + (colp + iota_f)
            c0 = jnp.minimum(jnp.maximum(c0, 1.0), jnp.float32(W))
            c1 = jnp.minimum(jnp.maximum(c1, 1.0), jnp.float32(H))
            l0 = c0.astype(jnp.int32)
            l1 = c1.astype(jnp.int32)
            l0f = l0.astype(jnp.float32)
            l1f = l1.astype(jnp.float32)
            r0 = l0 + (c0 > l0f).astype(jnp.int32)
            r1 = l1 + (c1 > l1f).astype(jnp.int32)
            f0 = c0 - l0f
            f1 = c1 - l1f
            lt = plsc.load_gather(plane_v, [l0, l1])
            rb = plsc.load_gather(plane_v, [r0, r1])
            lb = plsc.load_gather(plane_v, [l0, r1])
            rt = plsc.load_gather(plane_v, [r0, l1])
            vt = lt + (rt - lt) * f0
            vb = lb + (rb - lb) * f0
            rbuf[pl.ds(v * 16, 16)] = vt + (vb - vt) * f1
            return 0

    def per_plane(t, _):
        i = wid * PLANES_PER_W + t
        bb = i // C
        jj = i % C

        def offs_src(k):
            return offs_hbm.at[bb, 2 * jj + k // 4,
                               pl.ds((k % 4) * (2 * RPC), 2 * RPC),
                               pl.ds(0, W)]

        pltpu.sync_copy(planes_hbm.at[i], plane_v)
        pltpu.async_copy(offs_src(0), offs_v.at[0], sem_in)

        def pair(k2, _):
            for par in (0, 1):
                k = k2 + par
                obuf = offs_v.at[par]
                rbuf = out_v.at[par]
                pltpu.make_async_copy(offs_src(k), obuf, sem_in).wait()

                @pl.when(k + 1 < NCHUNK)
                def _():
                    pltpu.async_copy(offs_src(k + 1), offs_v.at[1 - par],
                                     sem_in)

                @pl.when(k >= 2)
                def _():
                    pltpu.make_async_copy(
                        rbuf, out_hbm.at[i, pl.ds((k - 2) * PX, PX)], sem_o[par]
                    ).wait()

                compute_chunk(k, obuf, rbuf)
                pltpu.async_copy(
                    rbuf, out_hbm.at[i, pl.ds(k * PX, PX)], sem_o[par])
            return 0

        lax.fori_loop(0, NCHUNK // 2, lambda j, c: pair(2 * j, c), 0)
        pltpu.make_async_copy(
            out_v.at[0], out_hbm.at[i, pl.ds((NCHUNK - 2) * PX, PX)], sem_o[0]
        ).wait()
        pltpu.make_async_copy(
            out_v.at[1], out_hbm.at[i, pl.ds((NCHUNK - 1) * PX, PX)], sem_o[1]
        ).wait()
        return 0

    lax.fori_loop(0, PLANES_PER_W, per_plane, 0)


@functools.cache
def _sample():
    return pl.kernel(
        _sample_body,
        out_type=jax.ShapeDtypeStruct((NPLANES, HW), jnp.float32),
        mesh=plsc.VectorSubcoreMesh(core_axis_name="c", subcore_axis_name="s"),
        scratch_types=[
            pltpu.VMEM((PH + 2, PWC), jnp.float32),
            pltpu.VMEM((2, 2 * RPC, W), jnp.float32),
            pltpu.VMEM((2, PX), jnp.float32),
            pltpu.SemaphoreType.DMA,
            pltpu.SemaphoreType.DMA,
            pltpu.SemaphoreType.DMA,
        ],
        compiler_params=pltpu.CompilerParams(use_tc_tiling_on_sc=False,
                                             needs_layout_passes=False),
    )


def kernel(x, W_conv, b_conv):
    x_chw = jnp.transpose(x, (0, 3, 1, 2))
    x_wide = jnp.pad(x_chw, ((0, 0), (0, 0), (1, 1), (1, PWC - W - 1)))
    x_flat = jnp.pad(x_wide.reshape(B, C, PH * PWC),
                     ((0, 0), (0, 0), (0, FLAT_IN - PH * PWC)))
    w_t = jnp.transpose(W_conv, (0, 1, 3, 2)).reshape(9, CO, C)
    bias_col = b_conv.reshape(CO, 1)

    conv = _offsets_conv(x_flat, w_t, bias_col)
    conv4 = conv.reshape(B, CO, NSTEPS * PT // PWC, PWC)

    planes = x_flat.reshape(NPLANES, PH + 2, PWC)

    mapped = _sample()(planes, conv4)
    return jnp.transpose(mapped.reshape(B, C, H, W), (0, 2, 3, 1))


# speedup vs baseline: 8.0861x; 1.0477x over previous
import functools

import jax
import jax.numpy as jnp
from jax import lax
from jax.experimental import pallas as pl
from jax.experimental.pallas import tpu as pltpu
from jax.experimental.pallas import tpu_sc as plsc

B, H, W, C = 2, 224, 224, 96
CO = 2 * C
HW = H * W
PW = W + 2
PH = H + 2
PWC = 256
FLAT_VALID = H * PWC
PT = 6400
NSTEPS = 9
XT = PT + 2 * PWC + 128
FLAT_IN = (NSTEPS - 1) * PT + XT + 128

NPLANES = B * C
NWORKERS = 32
PLANES_PER_W = C // NWORKERS
PX = 6272
NCHUNK = HW // PX
NVEC = PX // 16
VPR = W // 16
RPC = PX // W


def _conv_body(x_ref, w_ref, b_ref, out_ref):
    s = pl.program_id(1)
    xt = x_ref[0, :, pl.ds(s * PT, XT)]
    acc = jnp.zeros((CO, PT), dtype=jnp.float32)
    for kx in range(3):
        xk = xt if kx == 0 else pltpu.roll(xt, XT - kx, 1)
        for ky in range(3):
            xs = xk[:, ky * PWC:ky * PWC + PT]
            wt = w_ref[ky * 3 + kx]
            acc = acc + lax.dot_general(
                wt, xs, (((1,), (0,)), ((), ())),
                preferred_element_type=jnp.float32,
                precision=lax.Precision.DEFAULT)
    out_ref[0] = acc + b_ref[...]


def _offsets_conv(x_flat, w_t, bias_col):
    return pl.pallas_call(
        _conv_body,
        grid=(1, NSTEPS),
        in_specs=[
            pl.BlockSpec((1, C, FLAT_IN), lambda b, s: (b, 0, 0)),
            pl.BlockSpec((9, CO, C), lambda b, s: (0, 0, 0)),
            pl.BlockSpec((CO, 1), lambda b, s: (0, 0)),
        ],
        out_specs=pl.BlockSpec((1, CO, PT), lambda b, s: (b, 0, s)),
        out_shape=jax.ShapeDtypeStruct((1, CO, NSTEPS * PT), jnp.float32),
        compiler_params=pltpu.CompilerParams(
            vmem_limit_bytes=100 * 1024 * 1024),
    )(x_flat, w_t, bias_col)


def _sample_body(planes_hbm, offs_hbm, out_hbm, plane_v, offs_v, out_v,
                 sem_in, sem_o0, sem_o1):
    wid = lax.axis_index("s") * 2 + lax.axis_index("c")
    iota = lax.iota(jnp.int32, 16)
    iota_f = iota.astype(jnp.float32)
    idx_even = iota * 2
    sem_o = (sem_o0, sem_o1)

    def compute_chunk(k, obuf, rbuf):
        @functools.partial(lax.fori_loop, 0, NVEC, init_val=0, unroll=14)
        def _vec(v, _carry):
            rowp = (k * RPC + v // VPR + 1).astype(jnp.float32)
            colp = ((v % VPR) * 16 + 1).astype(jnp.float32)
            srow = jnp.broadcast_to(v // 7, (16,))
            scol = (v % 7) * 32 + idx_even
            o0 = plsc.load_gather(obuf, [srow, scol])
            o1 = plsc.load_gather(obuf, [srow, scol + 1])
            c0 = o0 + rowp
            c1 = o1 + (colp + iota_f)
            c0 = jnp.minimum(jnp.maximum(c0, 1.0), jnp.float32(W))
            c1 = jnp.minimum(jnp.maximum(c1, 1.0), jnp.float32(H))
            l0 = c0.astype(jnp.int32)
            l1 = c1.astype(jnp.int32)
            l0f = l0.astype(jnp.float32)
            l1f = l1.astype(jnp.float32)
            r0 = l0 + (c0 > l0f).astype(jnp.int32)
            r1 = l1 + (c1 > l1f).astype(jnp.int32)
            f0 = c0 - l0f
            f1 = c1 - l1f
            lt = plsc.load_gather(plane_v, [l0, l1])
            rb = plsc.load_gather(plane_v, [r0, r1])
            lb = plsc.load_gather(plane_v, [l0, r1])
            rt = plsc.load_gather(plane_v, [r0, l1])
            vt = lt + (rt - lt) * f0
            vb = lb + (rb - lb) * f0
            rbuf[pl.ds(v * 16, 16)] = vt + (vb - vt) * f1
            return 0

    def per_plane(t, _):
        i = wid * PLANES_PER_W + t
        jj = i

        def offs_src(k):
            return offs_hbm.at[2 * jj + k // 4,
                               pl.ds((k % 4) * (2 * RPC), 2 * RPC),
                               pl.ds(0, W)]

        pltpu.sync_copy(planes_hbm.at[i], plane_v)
        pltpu.async_copy(offs_src(0), offs_v.at[0], sem_in)

        def pair(k2, _):
            for par in (0, 1):
                k = k2 + par
                obuf = offs_v.at[par]
                rbuf = out_v.at[par]
                pltpu.make_async_copy(offs_src(k), obuf, sem_in).wait()

                @pl.when(k + 1 < NCHUNK)
                def _():
                    pltpu.async_copy(offs_src(k + 1), offs_v.at[1 - par],
                                     sem_in)

                @pl.when(k >= 2)
                def _():
                    pltpu.make_async_copy(
                        rbuf, out_hbm.at[i, pl.ds((k - 2) * PX, PX)], sem_o[par]
                    ).wait()

                compute_chunk(k, obuf, rbuf)
                pltpu.async_copy(
                    rbuf, out_hbm.at[i, pl.ds(k * PX, PX)], sem_o[par])
            return 0

        lax.fori_loop(0, NCHUNK // 2, lambda j, c: pair(2 * j, c), 0)
        pltpu.make_async_copy(
            out_v.at[0], out_hbm.at[i, pl.ds((NCHUNK - 2) * PX, PX)], sem_o[0]
        ).wait()
        pltpu.make_async_copy(
            out_v.at[1], out_hbm.at[i, pl.ds((NCHUNK - 1) * PX, PX)], sem_o[1]
        ).wait()
        return 0

    lax.fori_loop(0, PLANES_PER_W, per_plane, 0)


@functools.cache
def _sample():
    return pl.kernel(
        _sample_body,
        out_type=jax.ShapeDtypeStruct((C, HW), jnp.float32),
        mesh=plsc.VectorSubcoreMesh(core_axis_name="c", subcore_axis_name="s"),
        scratch_types=[
            pltpu.VMEM((PH + 2, PWC), jnp.float32),
            pltpu.VMEM((2, 2 * RPC, W), jnp.float32),
            pltpu.VMEM((2, PX), jnp.float32),
            pltpu.SemaphoreType.DMA,
            pltpu.SemaphoreType.DMA,
            pltpu.SemaphoreType.DMA,
        ],
        compiler_params=pltpu.CompilerParams(use_tc_tiling_on_sc=False,
                                             needs_layout_passes=False),
    )


def kernel(x, W_conv, b_conv):
    x_chw = jnp.transpose(x, (0, 3, 1, 2))
    x_wide = jnp.pad(x_chw, ((0, 0), (0, 0), (1, 1), (1, PWC - W - 1)))
    x_flat = jnp.pad(x_wide.reshape(B, C, PH * PWC),
                     ((0, 0), (0, 0), (0, FLAT_IN - PH * PWC)))
    w_t = jnp.transpose(W_conv, (0, 1, 3, 2)).reshape(9, CO, C)
    bias_col = b_conv.reshape(CO, 1)

    mapped = []
    for b in range(B):
        xb = x_flat[b:b + 1]
        convb = _offsets_conv(xb, w_t, bias_col)
        conv4b = convb.reshape(CO, NSTEPS * PT // PWC, PWC)
        planesb = xb.reshape(C, PH + 2, PWC)
        mapped.append(_sample()(planesb, conv4b))
    out = jnp.stack(mapped).reshape(B, C, H, W)
    return jnp.transpose(out, (0, 2, 3, 1))
